# Initial kernel scaffold; baseline (speedup 1.0000x reference)
import jax
import jax.numpy as jnp
from jax.experimental import pallas as pl


def kernel(x, agg_fc_w, agg_fc_b, mh_w, mh_b, shrink_w, shrink_b, nearest_nodes):
    raise NotImplementedError("write your pallas kernel here")


# speedup vs baseline: 13.1214x; 13.1214x over previous
import functools

import jax
import jax.numpy as jnp
from jax.experimental import pallas as pl
from jax.experimental.pallas import tpu as pltpu

_ALPHA = 0.2
_BETA = 0.8


def _gal_body(t_in, t_out, k1, nn_ref, aw_ref, ab_ref, w3_ref, cb_ref,
              x_ref, o_ref):
    xb = x_ref[0]
    n = xb.shape[1]
    o_ref[0, 0:t_in, :, :] = xb

    iota = jax.lax.broadcasted_iota(jnp.int32, (n, n), 1)
    a_mat = jnp.zeros((n, n), jnp.float32)
    for k in range(k1):
        sel = (nn_ref[:, k:k + 1] == iota).astype(jnp.float32)
        a_mat = a_mat + sel * aw_ref[0:1, k:k + 1]

    v = jax.lax.dot_general(a_mat, xb, (((1,), (1,)), ((), ())),
                            preferred_element_type=jnp.float32)
    v = v + ab_ref[...]
    s = v * jax.nn.sigmoid(_BETA * v)

    y = jax.lax.dot_general(w3_ref[...], s, (((1,), (1,)), ((), ())),
                            preferred_element_type=jnp.float32)
    y = y + cb_ref[...]
    selu_scale = 1.0507009873554805
    selu_alpha = 1.6732632423543772
    y = selu_scale * jnp.where(y > 0, y, selu_alpha * (jnp.exp(y) - 1.0))
    o_ref[0, t_in:t_in + t_out, :, :] = y


def kernel(x, agg_fc_w, agg_fc_b, mh_w, mh_b, shrink_w, shrink_b,
           nearest_nodes):
    b, t_in, n, c = x.shape
    t_extra = shrink_w.shape[0]
    heads = mh_w.shape[0]
    k1 = nearest_nodes.shape[1]

    sw = shrink_w.reshape(t_extra, t_in, heads)
    w2 = jnp.einsum('oth,h->ot', sw, mh_w[:, 0])
    decay = jnp.full((t_in,), 1.0 - _ALPHA).at[0].set(1.0)
    w3 = w2 * decay[None, :] + _ALPHA * jnp.pad(w2[:, 1:], ((0, 0), (0, 1)))
    const = shrink_b + jnp.einsum('oth,h->o', sw, mh_b)

    aw = agg_fc_w.astype(jnp.float32)
    ab = jnp.broadcast_to(agg_fc_b.reshape(1, 1, 1), (1, 1, c))
    cb = jnp.broadcast_to(const[:, None, None], (t_extra, 1, c))

    body = functools.partial(_gal_body, t_in, t_extra, k1)
    out = pl.pallas_call(
        body,
        grid=(b,),
        in_specs=[
            pl.BlockSpec((n, k1), lambda i: (0, 0)),
            pl.BlockSpec((1, k1), lambda i: (0, 0)),
            pl.BlockSpec((1, 1, c), lambda i: (0, 0, 0)),
            pl.BlockSpec((t_extra, t_in), lambda i: (0, 0)),
            pl.BlockSpec((t_extra, 1, c), lambda i: (0, 0, 0)),
            pl.BlockSpec((1, t_in, n, c), lambda i: (i, 0, 0, 0)),
        ],
        out_specs=pl.BlockSpec((1, t_in + t_extra, n, c),
                               lambda i: (i, 0, 0, 0)),
        out_shape=jax.ShapeDtypeStruct((b, t_in + t_extra, n, c),
                                       jnp.float32),
        compiler_params=pltpu.CompilerParams(
            dimension_semantics=("parallel",)),
    )(nearest_nodes, aw, ab, w3, cb, x)
    return out
